# baseline (speedup 1.0000x reference)
import functools

import jax
import jax.numpy as jnp
import numpy as np
from jax.experimental import pallas as pl
from jax.sharding import Mesh, PartitionSpec as P

try:
    from jax import shard_map as _shard_map_fn

    def _shard_map(f, mesh, in_specs, out_specs):
        return _shard_map_fn(f, mesh=mesh, in_specs=in_specs,
                             out_specs=out_specs, check_vma=False)
except ImportError:
    from jax.experimental.shard_map import shard_map as _shard_map_fn

    def _shard_map(f, mesh, in_specs, out_specs):
        return _shard_map_fn(f, mesh=mesh, in_specs=in_specs,
                             out_specs=out_specs, check_rep=False)


def _dist_block_kernel(x_ref, y_ref, out_ref):
    x = x_ref[...]
    y = y_ref[...]
    xn = jnp.sum(x * x, axis=1, keepdims=True)
    yn = jnp.sum(y * y, axis=1, keepdims=True)
    dot = jax.lax.dot_general(
        x, y, (((1,), (1,)), ((), ())),
        preferred_element_type=jnp.float32,
    )
    out_ref[...] = (xn - 2.0 * dot) + yn.T


@functools.partial(jax.jit, static_argnames=("bq", "bk"))
def _dist(x, y, bq, bk):
    q, d = x.shape
    k, _ = y.shape
    grid = (q // bq, k // bk)
    return pl.pallas_call(
        _dist_block_kernel,
        grid=grid,
        in_specs=[
            pl.BlockSpec((bq, d), lambda i, j: (i, 0)),
            pl.BlockSpec((bk, d), lambda i, j: (j, 0)),
        ],
        out_specs=pl.BlockSpec((bq, bk), lambda i, j: (i, j)),
        out_shape=jax.ShapeDtypeStruct((q, k), jnp.float32),
    )(x, y)


def kernel(x, y):
    devs = jax.devices()
    ndev = len(devs)
    k = y.shape[0]
    if ndev > 1 and k % ndev == 0:
        mesh = Mesh(np.array(devs), ("k",))
        bk = min(4096, k // ndev)
        f = _shard_map(
            lambda xs, ys: _dist(xs, ys, bq=1024, bk=bk),
            mesh=mesh, in_specs=(P(), P("k")), out_specs=P(None, "k"),
        )
        return f(x, y)
    return _dist(x, y, bq=1024, bk=4096)


# speedup vs baseline: 10.8814x; 10.8814x over previous
import functools

import jax
import jax.numpy as jnp
from jax.experimental import pallas as pl
from jax.experimental.pallas import tpu as pltpu


def _dist_block_kernel(x_ref, y_ref, out_ref):
    x = x_ref[...]
    y = y_ref[...]
    xn = jnp.sum(x * x, axis=1, keepdims=True)
    yn = jnp.sum(y * y, axis=1, keepdims=True)
    dot = jax.lax.dot_general(
        x, y, (((1,), (1,)), ((), ())),
        preferred_element_type=jnp.float32,
    )
    out_ref[...] = (xn - 2.0 * dot) + yn.T


@functools.partial(jax.jit, static_argnames=("bq", "bk"))
def _dist(x, y, bq, bk):
    q, d = x.shape
    k, _ = y.shape
    grid = (q // bq, k // bk)
    return pl.pallas_call(
        _dist_block_kernel,
        grid=grid,
        in_specs=[
            pl.BlockSpec((bq, d), lambda i, j: (i, 0)),
            pl.BlockSpec((bk, d), lambda i, j: (j, 0)),
        ],
        out_specs=pl.BlockSpec((bq, bk), lambda i, j: (i, j)),
        out_shape=jax.ShapeDtypeStruct((q, k), jnp.float32),
        compiler_params=pltpu.CompilerParams(
            dimension_semantics=("parallel", "parallel"),
        ),
    )(x, y)


def kernel(x, y):
    return _dist(x, y, bq=1024, bk=4096)
